# Initial kernel scaffold; baseline (speedup 1.0000x reference)
import jax
import jax.numpy as jnp
from jax.experimental import pallas as pl


def kernel(x, edge_index, W, b):
    raise NotImplementedError("write your pallas kernel here")


# speedup vs baseline: 29.0030x; 29.0030x over previous
import functools

import jax
import jax.numpy as jnp
from jax import lax
from jax.experimental import pallas as pl
from jax.experimental.pallas import tpu as pltpu
from jax.experimental.pallas import tpu_sc as plsc

NC = 2
NS = 16
NW = NC * NS
CH = 125


def _deg_body(rows_per_w, dst2d, zeros1, deg0_out, deg1_out,
              idx_v, ones_v, deg_sh):
  cid = lax.axis_index("c")
  sid = lax.axis_index("s")
  w = cid * NS + sid

  for i in range(8):
    ones_v[pl.ds(i * 16, 16)] = jnp.ones((16,), jnp.float32)

  @pl.when(sid == 0)
  def _():
    pltpu.sync_copy(zeros1, deg_sh)

  plsc.subcore_barrier()
  pltpu.sync_copy(dst2d.at[pl.ds(w * rows_per_w, rows_per_w)], idx_v)

  def step(j, carry):
    pltpu.sync_copy(ones_v.at[pl.ds(0, CH)], deg_sh.at[idx_v.at[j]], add=True)
    return carry

  lax.fori_loop(0, rows_per_w, step, 0)
  plsc.subcore_barrier()

  @pl.when((sid == 0) & (cid == 0))
  def _():
    pltpu.sync_copy(deg_sh, deg0_out)

  @pl.when((sid == 0) & (cid == 1))
  def _():
    pltpu.sync_copy(deg_sh, deg1_out)


def _deg_call(dst2d, zeros1, npad):
  rows = dst2d.shape[0]
  assert rows % NW == 0
  rows_per_w = rows // NW
  mesh = plsc.VectorSubcoreMesh(core_axis_name="c", subcore_axis_name="s")
  return pl.kernel(
      functools.partial(_deg_body, rows_per_w),
      out_type=[
          jax.ShapeDtypeStruct((npad,), jnp.float32),
          jax.ShapeDtypeStruct((npad,), jnp.float32),
      ],
      mesh=mesh,
      scratch_types=[
          pltpu.VMEM((rows_per_w, CH), jnp.int32),
          pltpu.VMEM((128,), jnp.float32),
          pltpu.VMEM_SHARED((npad,), jnp.float32),
      ],
  )(dst2d, zeros1)


def _mm_body(x_ref, w_ref, d0_ref, d1_ref, y_ref, dis_ref):
  xw = jnp.dot(x_ref[...], w_ref[...], preferred_element_type=jnp.float32)
  deg = d0_ref[...] + d1_ref[...] + 1.0
  dis = lax.rsqrt(deg)
  y_ref[...] = xw * dis
  dis_ref[...] = dis


def _mm_call(x, w, d0, d1):
  n, din = x.shape
  dout = w.shape[1]
  blk = 512
  grid = n // blk
  return pl.pallas_call(
      _mm_body,
      grid=(grid,),
      in_specs=[
          pl.BlockSpec((blk, din), lambda i: (i, 0)),
          pl.BlockSpec((din, dout), lambda i: (0, 0)),
          pl.BlockSpec((blk, 1), lambda i: (i, 0)),
          pl.BlockSpec((blk, 1), lambda i: (i, 0)),
      ],
      out_specs=[
          pl.BlockSpec((blk, dout), lambda i: (i, 0)),
          pl.BlockSpec((blk, 1), lambda i: (i, 0)),
      ],
      out_shape=[
          jax.ShapeDtypeStruct((n, dout), jnp.float32),
          jax.ShapeDtypeStruct((n, 1), jnp.float32),
      ],
  )(x, w, d0, d1)


def _agg_body(npad, rows_per_w, y_hbm, src2d, dst2d, zeros2d, p_out,
              sidx_v, didx_v, rows_v, acc_sh, sem):
  cid = lax.axis_index("c")
  sid = lax.axis_index("s")
  w = cid * NS + sid
  rpt = npad // NS

  @pl.when(cid == 0)
  def _():
    pltpu.sync_copy(y_hbm.at[pl.ds(sid * rpt, rpt)],
                    acc_sh.at[pl.ds(sid * rpt, rpt)])

  @pl.when(cid == 1)
  def _():
    pltpu.sync_copy(zeros2d.at[pl.ds(sid * rpt, rpt)],
                    acc_sh.at[pl.ds(sid * rpt, rpt)])

  pltpu.sync_copy(src2d.at[pl.ds(w * rows_per_w, rows_per_w)], sidx_v)
  pltpu.sync_copy(dst2d.at[pl.ds(w * rows_per_w, rows_per_w)], didx_v)
  plsc.subcore_barrier()

  def step(j, carry):
    pltpu.async_copy(y_hbm.at[sidx_v.at[j]], rows_v, sem).wait()
    pltpu.sync_copy(rows_v, acc_sh.at[didx_v.at[j]], add=True)
    return carry

  lax.fori_loop(0, rows_per_w, step, 0)
  plsc.subcore_barrier()

  pltpu.sync_copy(acc_sh.at[pl.ds(sid * rpt, rpt)],
                  p_out.at[cid, pl.ds(sid * rpt, rpt)])


def _agg_call(y, src2d, dst2d, zeros2d):
  npad, d = y.shape
  rows = src2d.shape[0]
  assert rows % NW == 0 and npad % NS == 0
  rows_per_w = rows // NW
  mesh = plsc.VectorSubcoreMesh(core_axis_name="c", subcore_axis_name="s")
  return pl.kernel(
      functools.partial(_agg_body, npad, rows_per_w),
      out_type=jax.ShapeDtypeStruct((NC, npad, d), jnp.float32),
      mesh=mesh,
      scratch_types=[
          pltpu.VMEM((rows_per_w, CH), jnp.int32),
          pltpu.VMEM((rows_per_w, CH), jnp.int32),
          pltpu.VMEM((CH, d), jnp.float32),
          pltpu.VMEM_SHARED((npad, d), jnp.float32),
          pltpu.SemaphoreType.DMA,
      ],
  )(y, src2d, dst2d, zeros2d)


def _fin_body(p_ref, dis_ref, b_ref, o_ref):
  acc = p_ref[0] + p_ref[1]
  o_ref[...] = jnp.maximum(dis_ref[...] * acc + b_ref[...], 0.0)


def _fin_call(p, dis, b2d, n):
  _, npad, d = p.shape
  blk = 400
  grid = n // blk
  return pl.pallas_call(
      _fin_body,
      grid=(grid,),
      in_specs=[
          pl.BlockSpec((NC, blk, d), lambda i: (0, i, 0)),
          pl.BlockSpec((blk, 1), lambda i: (i, 0)),
          pl.BlockSpec((1, d), lambda i: (0, 0)),
      ],
      out_specs=pl.BlockSpec((blk, d), lambda i: (i, 0)),
      out_shape=jax.ShapeDtypeStruct((n, d), jnp.float32),
  )(p, dis, b2d)


@jax.jit
def kernel(x, edge_index, W, b):
  n, din = x.shape
  e = edge_index.shape[1]
  dout = W.shape[1]
  assert e % (NW * CH) == 0
  npad = ((n + 511) // 512) * 512

  xp = jnp.pad(x, ((0, npad - n), (0, 0)))
  src2d = edge_index[0].reshape(e // CH, CH)
  dst2d = edge_index[1].reshape(e // CH, CH)
  zeros1 = jnp.zeros((npad,), jnp.float32)
  zeros2d = jnp.zeros((npad, dout), jnp.float32)

  deg0, deg1 = _deg_call(dst2d, zeros1, npad)
  y, dis = _mm_call(xp, W, deg0.reshape(npad, 1), deg1.reshape(npad, 1))
  p = _agg_call(y, src2d, dst2d, zeros2d)
  return _fin_call(p, dis, b.reshape(1, -1), n)


# speedup vs baseline: 38.0588x; 1.3122x over previous
import functools

import jax
import jax.numpy as jnp
from jax import lax
from jax.experimental import pallas as pl
from jax.experimental.pallas import tpu as pltpu
from jax.experimental.pallas import tpu_sc as plsc

NC = 2
NS = 16
NW = NC * NS
CH = 125


def _deg_body(rows_per_w, e3, zeros1, deg0_out, deg1_out,
              idx_v, ones_v, deg_sh):
  cid = lax.axis_index("c")
  sid = lax.axis_index("s")
  w = cid * NS + sid

  for i in range(8):
    ones_v[pl.ds(i * 16, 16)] = jnp.ones((16,), jnp.float32)

  @pl.when(sid == 0)
  def _():
    pltpu.sync_copy(zeros1, deg_sh)

  plsc.subcore_barrier()
  pltpu.sync_copy(e3.at[1, pl.ds(w * rows_per_w, rows_per_w)], idx_v)

  def step(j, carry):
    pltpu.sync_copy(ones_v.at[pl.ds(0, CH)], deg_sh.at[idx_v.at[j]], add=True)
    return carry

  lax.fori_loop(0, rows_per_w, step, 0)
  plsc.subcore_barrier()

  @pl.when((sid == 0) & (cid == 0))
  def _():
    pltpu.sync_copy(deg_sh, deg0_out)

  @pl.when((sid == 0) & (cid == 1))
  def _():
    pltpu.sync_copy(deg_sh, deg1_out)


def _deg_call(e3, zeros1, npad):
  rows = e3.shape[1]
  assert rows % NW == 0
  rows_per_w = rows // NW
  mesh = plsc.VectorSubcoreMesh(core_axis_name="c", subcore_axis_name="s")
  return pl.kernel(
      functools.partial(_deg_body, rows_per_w),
      out_type=[
          jax.ShapeDtypeStruct((npad,), jnp.float32),
          jax.ShapeDtypeStruct((npad,), jnp.float32),
      ],
      mesh=mesh,
      scratch_types=[
          pltpu.VMEM((rows_per_w, CH), jnp.int32),
          pltpu.VMEM((128,), jnp.float32),
          pltpu.VMEM_SHARED((npad,), jnp.float32),
      ],
  )(e3, zeros1)


def _mm_body(x_ref, w_ref, d0_ref, d1_ref, y_ref, dis_ref):
  xw = jnp.dot(x_ref[...], w_ref[...], preferred_element_type=jnp.float32)
  deg = d0_ref[...] + d1_ref[...] + 1.0
  dis = lax.rsqrt(deg)
  y_ref[...] = xw * dis
  dis_ref[...] = dis


def _mm_call(x, w, d0, d1):
  n, din = x.shape
  dout = w.shape[1]
  blk = 512
  grid = n // blk
  return pl.pallas_call(
      _mm_body,
      grid=(grid,),
      in_specs=[
          pl.BlockSpec((blk, din), lambda i: (i, 0)),
          pl.BlockSpec((din, dout), lambda i: (0, 0)),
          pl.BlockSpec((blk, 1), lambda i: (i, 0)),
          pl.BlockSpec((blk, 1), lambda i: (i, 0)),
      ],
      out_specs=[
          pl.BlockSpec((blk, dout), lambda i: (i, 0)),
          pl.BlockSpec((blk, 1), lambda i: (i, 0)),
      ],
      out_shape=[
          jax.ShapeDtypeStruct((n, dout), jnp.float32),
          jax.ShapeDtypeStruct((n, 1), jnp.float32),
      ],
  )(x, w, d0, d1)


GRP = 8


def _agg_body(npad, rows_per_w, y_hbm, e3, zeros2d, p_out,
              sa_s, sa_d, sb_s, sb_d, rows0, rows1, acc_sh,
              sg0, sg1, si0, si1):
  rows_bufs = (rows0, rows1)
  gsems = (sg0, sg1)
  cid = lax.axis_index("c")
  sid = lax.axis_index("s")
  w = cid * NS + sid
  rpt = npad // NS
  base_row = w * rows_per_w
  n_sg = rows_per_w // GRP

  def process(src_i, dst_i):
    cps = [None, None]
    cps[0] = pltpu.async_copy(y_hbm.at[src_i.at[0]], rows_bufs[0], gsems[0])
    for k in range(GRP):
      nb = (k + 1) % 2
      if k + 1 < GRP:
        cps[nb] = pltpu.async_copy(
            y_hbm.at[src_i.at[k + 1]], rows_bufs[nb], gsems[nb])
      cps[k % 2].wait()
      pltpu.sync_copy(rows_bufs[k % 2], acc_sh.at[dst_i.at[k]], add=True)

  @pl.when(cid == 0)
  def _():
    pltpu.sync_copy(y_hbm.at[pl.ds(sid * rpt, rpt)],
                    acc_sh.at[pl.ds(sid * rpt, rpt)])

  @pl.when(cid == 1)
  def _():
    pltpu.sync_copy(zeros2d.at[pl.ds(sid * rpt, rpt)],
                    acc_sh.at[pl.ds(sid * rpt, rpt)])

  pltpu.sync_copy(e3.at[0, pl.ds(base_row, GRP)], sa_s)
  pltpu.sync_copy(e3.at[1, pl.ds(base_row, GRP)], sa_d)
  plsc.subcore_barrier()

  def body(i, carry):
    b0 = base_row + 2 * i * GRP

    pltpu.async_copy(e3.at[0, pl.ds(b0 + GRP, GRP)], sb_s, si0)
    pltpu.async_copy(e3.at[1, pl.ds(b0 + GRP, GRP)], sb_d, si1)
    process(sa_s, sa_d)
    pltpu.make_async_copy(e3.at[0, pl.ds(b0 + GRP, GRP)], sb_s, si0).wait()
    pltpu.make_async_copy(e3.at[1, pl.ds(b0 + GRP, GRP)], sb_d, si1).wait()

    nxt = b0 + 2 * GRP

    @pl.when(i + 1 < n_sg // 2)
    def _():
      pltpu.async_copy(e3.at[0, pl.ds(nxt, GRP)], sa_s, si0)
      pltpu.async_copy(e3.at[1, pl.ds(nxt, GRP)], sa_d, si1)

    process(sb_s, sb_d)

    @pl.when(i + 1 < n_sg // 2)
    def _():
      pltpu.make_async_copy(e3.at[0, pl.ds(nxt, GRP)], sa_s, si0).wait()
      pltpu.make_async_copy(e3.at[1, pl.ds(nxt, GRP)], sa_d, si1).wait()

    return carry

  assert n_sg % 2 == 0
  lax.fori_loop(0, n_sg // 2, body, 0)
  plsc.subcore_barrier()

  pltpu.sync_copy(acc_sh.at[pl.ds(sid * rpt, rpt)],
                  p_out.at[cid, pl.ds(sid * rpt, rpt)])


def _agg_call(y, e3, zeros2d):
  npad, d = y.shape
  rows = e3.shape[1]
  assert rows % NW == 0 and npad % NS == 0
  rows_per_w = rows // NW
  mesh = plsc.VectorSubcoreMesh(core_axis_name="c", subcore_axis_name="s")
  return pl.kernel(
      functools.partial(_agg_body, npad, rows_per_w),
      out_type=jax.ShapeDtypeStruct((NC, npad, d), jnp.float32),
      mesh=mesh,
      scratch_types=[
          pltpu.VMEM((GRP, CH), jnp.int32),
          pltpu.VMEM((GRP, CH), jnp.int32),
          pltpu.VMEM((GRP, CH), jnp.int32),
          pltpu.VMEM((GRP, CH), jnp.int32),
          pltpu.VMEM((CH, d), jnp.float32),
          pltpu.VMEM((CH, d), jnp.float32),
          pltpu.VMEM_SHARED((npad, d), jnp.float32),
          pltpu.SemaphoreType.DMA,
          pltpu.SemaphoreType.DMA,
          pltpu.SemaphoreType.DMA,
          pltpu.SemaphoreType.DMA,
      ],
  )(y, e3, zeros2d)


def _fin_body(p_ref, dis_ref, b_ref, o_ref):
  acc = p_ref[0] + p_ref[1]
  o_ref[...] = jnp.maximum(dis_ref[...] * acc + b_ref[...], 0.0)


def _fin_call(p, dis, b2d, n):
  _, npad, d = p.shape
  blk = 400
  grid = n // blk
  return pl.pallas_call(
      _fin_body,
      grid=(grid,),
      in_specs=[
          pl.BlockSpec((NC, blk, d), lambda i: (0, i, 0)),
          pl.BlockSpec((blk, 1), lambda i: (i, 0)),
          pl.BlockSpec((1, d), lambda i: (0, 0)),
      ],
      out_specs=pl.BlockSpec((blk, d), lambda i: (i, 0)),
      out_shape=jax.ShapeDtypeStruct((n, d), jnp.float32),
  )(p, dis, b2d)


@jax.jit
def kernel(x, edge_index, W, b):
  n, din = x.shape
  e = edge_index.shape[1]
  dout = W.shape[1]
  assert e % (NW * CH) == 0
  npad = ((n + 511) // 512) * 512

  xp = jnp.pad(x, ((0, npad - n), (0, 0)))
  e3 = edge_index.reshape(2, e // CH, CH)
  zeros1 = jnp.zeros((npad,), jnp.float32)
  zeros2d = jnp.zeros((npad, dout), jnp.float32)

  deg0, deg1 = _deg_call(e3, zeros1, npad)
  y, dis = _mm_call(xp, W, deg0.reshape(npad, 1), deg1.reshape(npad, 1))
  p = _agg_call(y, e3, zeros2d)
  return _fin_call(p, dis, b.reshape(1, -1), n)
